# baseline (speedup 1.0000x reference)
import functools

import jax
import jax.numpy as jnp
from jax import lax
from jax.experimental import pallas as pl
from jax.experimental.pallas import tpu as pltpu
from jax.experimental.pallas import tpu_sc as plsc

_NU, _NB, _NI = 20000, 10000, 20000
_D = 128
_E = 500000
_N_UB, _N_UI, _N_BI = _NU + _NB, _NU + _NI, _NB + _NI
_NS = (_N_UB, _N_UI, _N_BI)

_TB = 256
_EPAD = _TB * 128 * 16
_ER = _EPAD // 128

_CH = 4096
_NCH = (8, 10, 8)

_OFF = (0, _N_UB, _N_UB + _N_UI)
_DEGTOT = 100352
_DTB = (3 * _EPAD) // (128 * 32)
_DSL = _DEGTOT // 16


def _sc_mesh():
    return plsc.VectorSubcoreMesh(
        core_axis_name="c", subcore_axis_name="s", num_cores=2, num_subcores=16
    )


def _deg_call(dall):
    @functools.partial(
        pl.kernel,
        out_type=jax.ShapeDtypeStruct((2, _DEGTOT), jnp.float32),
        mesh=_sc_mesh(),
        scratch_types=[
            pltpu.VMEM((_DTB, 128), jnp.int32),
            pltpu.VMEM((128,), jnp.float32),
            pltpu.VMEM((784,), jnp.float32),
            pltpu.VMEM_SHARED((_DEGTOT,), jnp.float32),
            pltpu.SemaphoreType.DMA,
        ],
    )
    def deg_kernel(dall_ref, out_ref, idxv, ones_v, zer_v, acc, sem):
        c = lax.axis_index("c")
        s = lax.axis_index("s")
        w = c * 16 + s

        @pl.loop(0, 8)
        def _ones(i):
            ones_v[pl.ds(i * 16, 16)] = jnp.ones((16,), jnp.float32)

        @pl.loop(0, 49)
        def _zer(i):
            zer_v[pl.ds(i * 16, 16)] = jnp.zeros((16,), jnp.float32)

        for k in range(8):
            pltpu.sync_copy(zer_v, acc.at[pl.ds(s * _DSL + k * 784, 784)])
        plsc.subcore_barrier()

        pltpu.sync_copy(dall_ref.at[pl.ds(w * _DTB, _DTB)], idxv)

        @pl.loop(0, _DTB, step=4)
        def _hist(j0):
            descs = [
                pltpu.async_copy(
                    ones_v,
                    acc.at[plsc.Indices(idxv.at[j0 + b], ignored_value=-1)],
                    sem,
                    add=True,
                )
                for b in range(4)
            ]
            for d in descs:
                d.wait()

        plsc.subcore_barrier()
        pltpu.sync_copy(
            acc.at[pl.ds(s * _DSL, _DSL)], out_ref.at[c, pl.ds(s * _DSL, _DSL)]
        )

    return deg_kernel(dall)


def _chunk_rows(wrows, k):
    return max(0, min(wrows, (k + 1) * _CH) - k * _CH)


def _make_agg(wrows):
    nch = tuple(min(_NCH[g], -(-wrows[g] // _CH)) for g in range(3))
    gstart = (0, nch[0] % 2, (nch[0] + nch[1]) % 2)
    scratch = (
        [pltpu.VMEM((64, 128), jnp.int32), pltpu.VMEM((64, 128), jnp.int32)]
        + [pltpu.VMEM((128, _D), jnp.float32) for _ in range(4)]
        + [
            pltpu.VMEM((64, _D), jnp.float32),
            pltpu.VMEM_SHARED((_CH, _D), jnp.float32),
            pltpu.SemaphoreType.DMA,
            pltpu.SemaphoreType.DMA,
            pltpu.SemaphoreType.DMA,
            pltpu.SemaphoreType.DMA,
        ]
    )

    @functools.partial(
        pl.kernel,
        out_type=tuple(
            jax.ShapeDtypeStruct((n, _D), jnp.float32) for n in _NS
        ),
        mesh=_sc_mesh(),
        scratch_types=scratch,
    )
    def agg_kernel(ht0, ht1, ht2, si0, si1, si2, di0, di1, di2,
                   o0, o1, o2, srcv, dstv, *rest):
        rows = rest[:4]
        zz, acc = rest[4], rest[5]
        gsems = rest[6:8]
        ssems = rest[8:10]
        c = lax.axis_index("c")
        s = lax.axis_index("s")

        @pl.loop(0, 64)
        def _zfill(r):
            for cc in range(8):
                zz[r, pl.ds(cc * 16, 16)] = jnp.zeros((16,), jnp.float32)

        hts = (ht0, ht1, ht2)
        sis = (si0, si1, si2)
        dis = (di0, di1, di2)
        outs = (o0, o1, o2)

        def window(g, k, w0):
            ht = hts[g]
            pltpu.sync_copy(sis[g].at[k, pl.ds(s * _TB + w0, 64)], srcv)
            pltpu.sync_copy(dis[g].at[k, pl.ds(s * _TB + w0, 64)], dstv)

            @pl.loop(0, 64, step=4)
            def _grp(j0):
                gds = [
                    pltpu.async_copy(
                        ht.at[plsc.Indices(srcv.at[j0 + b], ignored_value=-1)],
                        rows[b],
                        gsems[b % 2],
                    )
                    for b in range(4)
                ]
                for d in gds:
                    d.wait()
                sds = [
                    pltpu.async_copy(
                        rows[b],
                        acc.at[plsc.Indices(dstv.at[j0 + b], ignored_value=-1)],
                        ssems[b % 2],
                        add=True,
                    )
                    for b in range(4)
                ]
                for d in sds:
                    d.wait()

        def one_chunk(g, k, wrc):
            for z in range(4):
                pltpu.sync_copy(zz, acc.at[pl.ds(s * 256 + z * 64, 64)])
            plsc.subcore_barrier()

            @pl.loop(0, 4)
            def _win(wi):
                window(g, k, wi * 64)

            plsc.subcore_barrier()
            row0 = k * _CH
            if wrc == _CH:
                pltpu.sync_copy(
                    acc.at[pl.ds(s * 256, 256)],
                    outs[g].at[pl.ds(row0 + s * 256, 256)],
                )
            else:
                wr_main = ((wrc // 16 + 7) // 8) * 8
                wr_last = wrc - 15 * wr_main

                @pl.when(s < 15)
                def _():
                    pltpu.sync_copy(
                        acc.at[pl.ds(s * wr_main, wr_main)],
                        outs[g].at[pl.ds(row0 + s * wr_main, wr_main)],
                    )

                @pl.when(s == 15)
                def _():
                    pltpu.sync_copy(
                        acc.at[pl.ds(15 * wr_main, wr_last)],
                        outs[g].at[pl.ds(row0 + 15 * wr_main, wr_last)],
                    )

            plsc.subcore_barrier()

        for cc in range(2):
            @pl.when(c == cc)
            def _(_cc=cc):
                for g in range(3):
                    a = (_cc + gstart[g]) % 2
                    ks = list(range(a, nch[g], 2))
                    full = [k for k in ks if _chunk_rows(wrows[g], k) == _CH]
                    assert full == ks[:len(full)]
                    if full:
                        @pl.loop(0, len(full))
                        def _chunks(i, _g=g, _a=a):
                            one_chunk(_g, _a + 2 * i, _CH)
                    for k in ks[len(full):]:
                        one_chunk(g, k, _chunk_rows(wrows[g], k))

    return agg_kernel


def _head_gather(h_u, h_b, iu2, ib2):
    @functools.partial(
        pl.kernel,
        out_type=(
            jax.ShapeDtypeStruct((16384, _D), jnp.float32),
            jax.ShapeDtypeStruct((16384, _D), jnp.float32),
        ),
        mesh=_sc_mesh(),
        scratch_types=[
            pltpu.VMEM((128, 128), jnp.int32),
            pltpu.VMEM((128, 128), jnp.int32),
            pltpu.VMEM((128, 128), jnp.float32),
            pltpu.SemaphoreType.DMA,
        ],
    )
    def gather_kernel(hu_ref, hb_ref, iu_ref, ib_ref, zu_ref, zb_ref,
                      idxu, idxb, rbuf, sem):
        c = lax.axis_index("c")
        s = lax.axis_index("s")
        w = c * 16 + s
        pltpu.sync_copy(iu_ref, idxu)
        pltpu.sync_copy(ib_ref, idxb)
        for j in range(4):
            pltpu.async_copy(hu_ref.at[idxu.at[w * 4 + j]], rbuf, sem).wait()
            pltpu.sync_copy(rbuf, zu_ref.at[pl.ds(w * 512 + j * 128, 128)])
        for j in range(4):
            pltpu.async_copy(hb_ref.at[idxb.at[w * 4 + j]], rbuf, sem).wait()
            pltpu.sync_copy(rbuf, zb_ref.at[pl.ds(w * 512 + j * 128, 128)])

    return gather_kernel(h_u, h_b, iu2, ib2)


def _mm_scaled(x, w, dinv):
    n = x.shape[0]
    bn = 800 if n % 800 == 0 else 600

    def body(xr, wr, dr, outr):
        outr[...] = (
            jnp.dot(xr[...], wr[...], preferred_element_type=jnp.float32)
            * dr[...]
        )

    return pl.pallas_call(
        body,
        grid=(n // bn,),
        in_specs=[
            pl.BlockSpec((bn, _D), lambda i: (i, 0)),
            pl.BlockSpec((_D, _D), lambda i: (0, 0)),
            pl.BlockSpec((bn, 1), lambda i: (i, 0)),
        ],
        out_specs=pl.BlockSpec((bn, _D), lambda i: (i, 0)),
        out_shape=jax.ShapeDtypeStruct((n, _D), jnp.float32),
    )(x, w, dinv)


def _dinv_call(degs):
    def body(dr, outr):
        outr[...] = lax.rsqrt(dr[0] + dr[1] + 1.0)

    return pl.pallas_call(
        body,
        out_shape=jax.ShapeDtypeStruct((784, 128), jnp.float32),
    )(degs)


def _combine(sa, hta, dva, ba, oa, sb, htb, dvb, bb, ob, nr):
    bn = 400

    def body(sar, har, dar, bar, sbr, hbr, dbr, bbr, outr):
        ya = dar[...] * (sar[...] + har[...]) + bar[...]
        ya = jnp.where(ya > 0, ya, jnp.exp(ya) - 1.0)
        yb = dbr[...] * (sbr[...] + hbr[...]) + bbr[...]
        yb = jnp.where(yb > 0, yb, jnp.exp(yb) - 1.0)
        outr[...] = 0.5 * (ya + yb)

    def rows_a(i):
        return (i + oa // bn, 0)

    def rows_b(i):
        return (i + ob // bn, 0)

    return pl.pallas_call(
        body,
        grid=(nr // bn,),
        in_specs=[
            pl.BlockSpec((bn, _D), rows_a),
            pl.BlockSpec((bn, _D), rows_a),
            pl.BlockSpec((bn, 1), rows_a),
            pl.BlockSpec((1, _D), lambda i: (0, 0)),
            pl.BlockSpec((bn, _D), rows_b),
            pl.BlockSpec((bn, _D), rows_b),
            pl.BlockSpec((bn, 1), rows_b),
            pl.BlockSpec((1, _D), lambda i: (0, 0)),
        ],
        out_specs=pl.BlockSpec((bn, _D), lambda i: (i, 0)),
        out_shape=jax.ShapeDtypeStruct((nr, _D), jnp.float32),
    )(sa, hta, dva, ba, sb, htb, dvb, bb)


def _mlp(zu, zb, wu, wb, b1, w2, b2, w3, b3, woT, bo):
    bn = 512

    def body(zur, zbr, wur, wbr, b1r, w2r, b2r, w3r, b3r, wor, bor, outr):
        z = jnp.dot(zur[...], wur[...], preferred_element_type=jnp.float32)
        z = z + jnp.dot(zbr[...], wbr[...], preferred_element_type=jnp.float32)
        z = jnp.maximum(z + b1r[...], 0.0)
        z = jnp.maximum(
            jnp.dot(z, w2r[...], preferred_element_type=jnp.float32) + b2r[...],
            0.0,
        )
        z = jnp.maximum(
            jnp.dot(z, w3r[...], preferred_element_type=jnp.float32) + b3r[...],
            0.0,
        )
        outr[...] = (
            jnp.sum(z * wor[...], axis=1, keepdims=True) + bor[...]
        )

    return pl.pallas_call(
        body,
        grid=(16384 // bn,),
        in_specs=[
            pl.BlockSpec((bn, _D), lambda i: (i, 0)),
            pl.BlockSpec((bn, _D), lambda i: (i, 0)),
            pl.BlockSpec((_D, 64), lambda i: (0, 0)),
            pl.BlockSpec((_D, 64), lambda i: (0, 0)),
            pl.BlockSpec((1, 64), lambda i: (0, 0)),
            pl.BlockSpec((64, 32), lambda i: (0, 0)),
            pl.BlockSpec((1, 32), lambda i: (0, 0)),
            pl.BlockSpec((32, 16), lambda i: (0, 0)),
            pl.BlockSpec((1, 16), lambda i: (0, 0)),
            pl.BlockSpec((1, 16), lambda i: (0, 0)),
            pl.BlockSpec((1, 1), lambda i: (0, 0)),
        ],
        out_specs=pl.BlockSpec((bn, 1), lambda i: (i, 0)),
        out_shape=jax.ShapeDtypeStruct((16384, 1), jnp.float32),
    )(zu, zb, wu, wb, b1, w2, b2, w3, b3, woT, bo)


def _chunk_indices(e, n, nchunks):
    src = e[0].astype(jnp.int32)
    dst = e[1].astype(jnp.int32)
    pad = jnp.full((_EPAD - _E,), -1, jnp.int32)
    srcp = jnp.concatenate([src, pad])
    dstp = jnp.concatenate([dst, pad])
    si, di = [], []
    for k in range(nchunks):
        valid = (dstp >= k * _CH) & (dstp < (k + 1) * _CH)
        si.append(jnp.where(valid, srcp, -1))
        di.append(jnp.where(valid, dstp - k * _CH, -1))
    return (
        jnp.stack(si).reshape(nchunks, _ER, 128),
        jnp.stack(di).reshape(nchunks, _ER, 128),
        dstp,
    )


def kernel(x, users, bundles, edge_index_ub, edge_index_ui, edge_index_bi,
           emb, W1_ub, b1_ub, W1_ui, b1_ui, W1_bi, b1_bi,
           W2_ub, b2_ub, W2_ui, b2_ui, W2_bi, b2_bi,
           Wfc1, bfc1, Wfc2, bfc2, Wfc3, bfc3, Wout, bout):
    del x

    si0, di0, dp0 = _chunk_indices(edge_index_ub, _N_UB, _NCH[0])
    si1, di1, dp1 = _chunk_indices(edge_index_ui, _N_UI, _NCH[1])
    si2, di2, dp2 = _chunk_indices(edge_index_bi, _N_BI, _NCH[2])

    dall = jnp.concatenate([
        jnp.where(dp0 >= 0, dp0 + _OFF[0], -1),
        jnp.where(dp1 >= 0, dp1 + _OFF[1], -1),
        jnp.where(dp2 >= 0, dp2 + _OFF[2], -1),
    ]).reshape((3 * _EPAD) // 128, 128)

    degs = _deg_call(dall)
    dinv_all = _dinv_call(degs.reshape(2, 784, 128)).reshape(-1)
    dv = (
        dinv_all[_OFF[0]:_OFF[0] + _N_UB].reshape(-1, 1),
        dinv_all[_OFF[1]:_OFF[1] + _N_UI].reshape(-1, 1),
        dinv_all[_OFF[2]:_OFF[2] + _N_BI].reshape(-1, 1),
    )

    x_ub = emb[:_N_UB]
    x_ui = jnp.concatenate([emb[:_NU], emb[_N_UB:]])
    x_bi = emb[_NU:]

    agg1 = _make_agg((_N_UB, _N_UI, _N_BI))
    agg2 = _make_agg((_N_UB, _NU, _NB))

    ws = ((W1_ub, W1_ui, W1_bi), (W2_ub, W2_ui, W2_bi))
    bs = (
        (b1_ub.reshape(1, _D), b1_ui.reshape(1, _D), b1_bi.reshape(1, _D)),
        (b2_ub.reshape(1, _D), b2_ui.reshape(1, _D), b2_bi.reshape(1, _D)),
    )

    h_u = h_b = None
    for layer in range(2):
        ht = tuple(
            _mm_scaled(xg, wg, dg)
            for xg, wg, dg in zip((x_ub, x_ui, x_bi), ws[layer], dv)
        )
        agg = agg1 if layer == 0 else agg2
        s = agg(ht[0], ht[1], ht[2], si0, si1, si2, di0, di1, di2)
        b3_ = bs[layer]
        h_u = _combine(s[0], ht[0], dv[0], b3_[0], 0,
                       s[1], ht[1], dv[1], b3_[1], 0, _NU)
        h_b = _combine(s[0], ht[0], dv[0], b3_[0], _NU,
                       s[2], ht[2], dv[2], b3_[2], 0, _NB)
        if layer == 0:
            h_i = _combine(s[1], ht[1], dv[1], b3_[1], _NU,
                           s[2], ht[2], dv[2], b3_[2], _NB, _NI)
            x_ub = jnp.concatenate([h_u, h_b])
            x_ui = jnp.concatenate([h_u, h_i])
            x_bi = jnp.concatenate([h_b, h_i])

    iu2 = users.reshape(-1).astype(jnp.int32).reshape(128, 128)
    ib2 = (bundles.reshape(-1).astype(jnp.int32) - _NU).reshape(128, 128)
    zu, zb = _head_gather(h_u, h_b, iu2, ib2)

    return _mlp(
        zu, zb, Wfc1[:_D], Wfc1[_D:], bfc1.reshape(1, 64),
        Wfc2, bfc2.reshape(1, 32), Wfc3, bfc3.reshape(1, 16),
        Wout.reshape(1, 16), bout.reshape(1, 1),
    )


# speedup vs baseline: 2.1461x; 2.1461x over previous
import functools

import jax
import jax.numpy as jnp
from jax import lax
from jax.experimental import pallas as pl
from jax.experimental.pallas import tpu as pltpu
from jax.experimental.pallas import tpu_sc as plsc

_NU, _NB, _NI = 20000, 10000, 20000
_D = 128
_E = 500000
_N_UB, _N_UI, _N_BI = _NU + _NB, _NU + _NI, _NB + _NI
_NS = (_N_UB, _N_UI, _N_BI)

_TB = 256
_EPAD = _TB * 128 * 16
_ER = _EPAD // 128

_CH = 4096
_NCH = (8, 10, 8)
_WIN = 4096

_OFF = (0, _N_UB, _N_UB + _N_UI)
_DEGTOT = 100352
_DTB = (3 * _EPAD) // (128 * 32)
_DSL = _DEGTOT // 16


def _sc_mesh():
    return plsc.VectorSubcoreMesh(
        core_axis_name="c", subcore_axis_name="s", num_cores=2, num_subcores=16
    )


def _deg_call(dall):
    @functools.partial(
        pl.kernel,
        out_type=jax.ShapeDtypeStruct((2, _DEGTOT), jnp.float32),
        mesh=_sc_mesh(),
        scratch_types=[
            pltpu.VMEM((_DTB, 128), jnp.int32),
            pltpu.VMEM((128,), jnp.float32),
            pltpu.VMEM((784,), jnp.float32),
            pltpu.VMEM_SHARED((_DEGTOT,), jnp.float32),
            pltpu.SemaphoreType.DMA,
        ],
    )
    def deg_kernel(dall_ref, out_ref, idxv, ones_v, zer_v, acc, sem):
        c = lax.axis_index("c")
        s = lax.axis_index("s")
        w = c * 16 + s

        @pl.loop(0, 8)
        def _ones(i):
            ones_v[pl.ds(i * 16, 16)] = jnp.ones((16,), jnp.float32)

        @pl.loop(0, 49)
        def _zer(i):
            zer_v[pl.ds(i * 16, 16)] = jnp.zeros((16,), jnp.float32)

        for k in range(8):
            pltpu.sync_copy(zer_v, acc.at[pl.ds(s * _DSL + k * 784, 784)])
        plsc.subcore_barrier()

        pltpu.sync_copy(dall_ref.at[pl.ds(w * _DTB, _DTB)], idxv)

        @pl.loop(0, _DTB, step=4)
        def _hist(j0):
            descs = [
                pltpu.async_copy(
                    ones_v,
                    acc.at[plsc.Indices(idxv.at[j0 + b], ignored_value=-1)],
                    sem,
                    add=True,
                )
                for b in range(4)
            ]
            for d in descs:
                d.wait()

        plsc.subcore_barrier()
        pltpu.sync_copy(
            acc.at[pl.ds(s * _DSL, _DSL)], out_ref.at[c, pl.ds(s * _DSL, _DSL)]
        )

    return deg_kernel(dall)


def _chunk_rows(wrows, k):
    return max(0, min(wrows, (k + 1) * _CH) - k * _CH)


def _make_agg(wrows):
    nch = tuple(min(_NCH[g], -(-wrows[g] // _CH)) for g in range(3))
    gstart = (0, nch[0] % 2, (nch[0] + nch[1]) % 2)
    scratch = (
        [pltpu.VMEM((_WIN,), jnp.int32), pltpu.VMEM((_WIN,), jnp.int32)]
        + [pltpu.VMEM((_WIN + 256,), jnp.int32) for _ in range(2)]
        + [pltpu.VMEM((128, _D), jnp.float32) for _ in range(2)]
        + [
            pltpu.VMEM((64, _D), jnp.float32),
            pltpu.VMEM_SHARED((_CH, _D), jnp.float32),
            pltpu.SemaphoreType.DMA,
            pltpu.SemaphoreType.DMA,
            pltpu.SemaphoreType.DMA,
            pltpu.SemaphoreType.DMA,
        ]
    )

    @functools.partial(
        pl.kernel,
        out_type=tuple(
            jax.ShapeDtypeStruct((n, _D), jnp.float32) for n in _NS
        ),
        mesh=_sc_mesh(),
        compiler_params=pltpu.CompilerParams(needs_layout_passes=False),
        scratch_types=scratch,
    )
    def agg_kernel(ht0, ht1, ht2, sf0, sf1, sf2, df0, df1, df2,
                   o0, o1, o2, srcb, dstb, csrc, cdst, *rest):
        rows = rest[:2]
        zz, acc = rest[2], rest[3]
        gsems = rest[4:6]
        ssems = rest[6:8]
        c = lax.axis_index("c")
        s = lax.axis_index("s")

        @pl.loop(0, 64)
        def _zfill(r):
            for cc in range(8):
                zz[r, pl.ds(cc * 16, 16)] = jnp.zeros((16,), jnp.float32)

        hts = (ht0, ht1, ht2)
        sfs = (sf0, sf1, sf2)
        dfs = (df0, df1, df2)
        outs = (o0, o1, o2)
        neg = jnp.full((16,), -1, jnp.int32)

        def window(g, k, w0):
            ht = hts[g]
            base = s * (_EPAD // 16) + w0
            pltpu.sync_copy(sfs[g].at[pl.ds(base, _WIN)], srcb)
            pltpu.sync_copy(dfs[g].at[pl.ds(base, _WIN)], dstb)
            lo = k * _CH

            @pl.loop(0, _WIN // 16, init_carry=jnp.int32(0))
            def cnt(i, cn):
                dv = dstb[pl.ds(i * 16, 16)]
                sv = srcb[pl.ds(i * 16, 16)]
                m = (dv >= lo) & (dv < lo + _CH)
                key = jnp.where(m, 0, 1).astype(jnp.uint32)
                _, sdv = plsc.sort_key_val(key, dv - lo)
                _, ssv = plsc.sort_key_val(key, sv)
                cdst[pl.ds(cn, 16)] = sdv
                csrc[pl.ds(cn, 16)] = ssv
                return cn + plsc.all_reduce_population_count(m)[0]

            @pl.loop(0, 16)
            def _pad(t):
                csrc[pl.ds(cnt + t * 16, 16)] = neg
                cdst[pl.ds(cnt + t * 16, 16)] = neg

            npair = (cnt + 255) // 256

            @pl.loop(0, _WIN // 256, step=1)
            def _st(p):
                @pl.when(p < npair)
                def _():
                    j = p * 2
                    g0 = pltpu.async_copy(
                        ht.at[plsc.Indices(csrc.at[pl.ds(j * 128, 128)],
                                           ignored_value=-1)],
                        rows[0], gsems[0])
                    g1 = pltpu.async_copy(
                        ht.at[plsc.Indices(csrc.at[pl.ds(j * 128 + 128, 128)],
                                           ignored_value=-1)],
                        rows[1], gsems[1])
                    g0.wait()
                    s0 = pltpu.async_copy(
                        rows[0],
                        acc.at[plsc.Indices(cdst.at[pl.ds(j * 128, 128)],
                                            ignored_value=-1)],
                        ssems[0], add=True)
                    g1.wait()
                    s1 = pltpu.async_copy(
                        rows[1],
                        acc.at[plsc.Indices(cdst.at[pl.ds(j * 128 + 128, 128)],
                                            ignored_value=-1)],
                        ssems[1], add=True)
                    s0.wait()
                    s1.wait()

        def one_chunk(g, k, wrc):
            for z in range(4):
                pltpu.sync_copy(zz, acc.at[pl.ds(s * 256 + z * 64, 64)])
            plsc.subcore_barrier()

            @pl.loop(0, _EPAD // 16 // _WIN)
            def _win(wi):
                window(g, k, wi * _WIN)

            plsc.subcore_barrier()
            row0 = k * _CH
            if wrc == _CH:
                pltpu.sync_copy(
                    acc.at[pl.ds(s * 256, 256)],
                    outs[g].at[pl.ds(row0 + s * 256, 256)],
                )
            else:
                wr_main = ((wrc // 16 + 7) // 8) * 8
                wr_last = wrc - 15 * wr_main

                @pl.when(s < 15)
                def _():
                    pltpu.sync_copy(
                        acc.at[pl.ds(s * wr_main, wr_main)],
                        outs[g].at[pl.ds(row0 + s * wr_main, wr_main)],
                    )

                @pl.when(s == 15)
                def _():
                    pltpu.sync_copy(
                        acc.at[pl.ds(15 * wr_main, wr_last)],
                        outs[g].at[pl.ds(row0 + 15 * wr_main, wr_last)],
                    )

            plsc.subcore_barrier()

        for cc in range(2):
            @pl.when(c == cc)
            def _(_cc=cc):
                for g in range(3):
                    a = (_cc + gstart[g]) % 2
                    ks = list(range(a, nch[g], 2))
                    full = [k for k in ks if _chunk_rows(wrows[g], k) == _CH]
                    assert full == ks[:len(full)]
                    if full:
                        @pl.loop(0, len(full))
                        def _chunks(i, _g=g, _a=a):
                            one_chunk(_g, _a + 2 * i, _CH)
                    for k in ks[len(full):]:
                        one_chunk(g, k, _chunk_rows(wrows[g], k))

    return agg_kernel


def _head_gather(h_u, h_b, iu2, ib2):
    @functools.partial(
        pl.kernel,
        out_type=(
            jax.ShapeDtypeStruct((16384, _D), jnp.float32),
            jax.ShapeDtypeStruct((16384, _D), jnp.float32),
        ),
        mesh=_sc_mesh(),
        scratch_types=[
            pltpu.VMEM((128, 128), jnp.int32),
            pltpu.VMEM((128, 128), jnp.int32),
            pltpu.VMEM((128, 128), jnp.float32),
            pltpu.SemaphoreType.DMA,
        ],
    )
    def gather_kernel(hu_ref, hb_ref, iu_ref, ib_ref, zu_ref, zb_ref,
                      idxu, idxb, rbuf, sem):
        c = lax.axis_index("c")
        s = lax.axis_index("s")
        w = c * 16 + s
        pltpu.sync_copy(iu_ref, idxu)
        pltpu.sync_copy(ib_ref, idxb)
        for j in range(4):
            pltpu.async_copy(hu_ref.at[idxu.at[w * 4 + j]], rbuf, sem).wait()
            pltpu.sync_copy(rbuf, zu_ref.at[pl.ds(w * 512 + j * 128, 128)])
        for j in range(4):
            pltpu.async_copy(hb_ref.at[idxb.at[w * 4 + j]], rbuf, sem).wait()
            pltpu.sync_copy(rbuf, zb_ref.at[pl.ds(w * 512 + j * 128, 128)])

    return gather_kernel(h_u, h_b, iu2, ib2)


def _mm_scaled(x, w, dinv):
    n = x.shape[0]
    bn = 800 if n % 800 == 0 else 600

    def body(xr, wr, dr, outr):
        outr[...] = (
            jnp.dot(xr[...], wr[...], preferred_element_type=jnp.float32)
            * dr[...]
        )

    return pl.pallas_call(
        body,
        grid=(n // bn,),
        in_specs=[
            pl.BlockSpec((bn, _D), lambda i: (i, 0)),
            pl.BlockSpec((_D, _D), lambda i: (0, 0)),
            pl.BlockSpec((bn, 1), lambda i: (i, 0)),
        ],
        out_specs=pl.BlockSpec((bn, _D), lambda i: (i, 0)),
        out_shape=jax.ShapeDtypeStruct((n, _D), jnp.float32),
    )(x, w, dinv)


def _dinv_call(degs):
    def body(dr, outr):
        outr[...] = lax.rsqrt(dr[0] + dr[1] + 1.0)

    return pl.pallas_call(
        body,
        out_shape=jax.ShapeDtypeStruct((784, 128), jnp.float32),
    )(degs)


def _combine(sa, hta, dva, ba, oa, sb, htb, dvb, bb, ob, nr):
    bn = 400

    def body(sar, har, dar, bar, sbr, hbr, dbr, bbr, outr):
        ya = dar[...] * (sar[...] + har[...]) + bar[...]
        ya = jnp.where(ya > 0, ya, jnp.exp(ya) - 1.0)
        yb = dbr[...] * (sbr[...] + hbr[...]) + bbr[...]
        yb = jnp.where(yb > 0, yb, jnp.exp(yb) - 1.0)
        outr[...] = 0.5 * (ya + yb)

    def rows_a(i):
        return (i + oa // bn, 0)

    def rows_b(i):
        return (i + ob // bn, 0)

    return pl.pallas_call(
        body,
        grid=(nr // bn,),
        in_specs=[
            pl.BlockSpec((bn, _D), rows_a),
            pl.BlockSpec((bn, _D), rows_a),
            pl.BlockSpec((bn, 1), rows_a),
            pl.BlockSpec((1, _D), lambda i: (0, 0)),
            pl.BlockSpec((bn, _D), rows_b),
            pl.BlockSpec((bn, _D), rows_b),
            pl.BlockSpec((bn, 1), rows_b),
            pl.BlockSpec((1, _D), lambda i: (0, 0)),
        ],
        out_specs=pl.BlockSpec((bn, _D), lambda i: (i, 0)),
        out_shape=jax.ShapeDtypeStruct((nr, _D), jnp.float32),
    )(sa, hta, dva, ba, sb, htb, dvb, bb)


def _mlp(zu, zb, wu, wb, b1, w2, b2, w3, b3, woT, bo):
    bn = 512

    def body(zur, zbr, wur, wbr, b1r, w2r, b2r, w3r, b3r, wor, bor, outr):
        z = jnp.dot(zur[...], wur[...], preferred_element_type=jnp.float32)
        z = z + jnp.dot(zbr[...], wbr[...], preferred_element_type=jnp.float32)
        z = jnp.maximum(z + b1r[...], 0.0)
        z = jnp.maximum(
            jnp.dot(z, w2r[...], preferred_element_type=jnp.float32) + b2r[...],
            0.0,
        )
        z = jnp.maximum(
            jnp.dot(z, w3r[...], preferred_element_type=jnp.float32) + b3r[...],
            0.0,
        )
        outr[...] = (
            jnp.sum(z * wor[...], axis=1, keepdims=True) + bor[...]
        )

    return pl.pallas_call(
        body,
        grid=(16384 // bn,),
        in_specs=[
            pl.BlockSpec((bn, _D), lambda i: (i, 0)),
            pl.BlockSpec((bn, _D), lambda i: (i, 0)),
            pl.BlockSpec((_D, 64), lambda i: (0, 0)),
            pl.BlockSpec((_D, 64), lambda i: (0, 0)),
            pl.BlockSpec((1, 64), lambda i: (0, 0)),
            pl.BlockSpec((64, 32), lambda i: (0, 0)),
            pl.BlockSpec((1, 32), lambda i: (0, 0)),
            pl.BlockSpec((32, 16), lambda i: (0, 0)),
            pl.BlockSpec((1, 16), lambda i: (0, 0)),
            pl.BlockSpec((1, 16), lambda i: (0, 0)),
            pl.BlockSpec((1, 1), lambda i: (0, 0)),
        ],
        out_specs=pl.BlockSpec((bn, 1), lambda i: (i, 0)),
        out_shape=jax.ShapeDtypeStruct((16384, 1), jnp.float32),
    )(zu, zb, wu, wb, b1, w2, b2, w3, b3, woT, bo)


def _pad_flat(e):
    pad = jnp.full((_EPAD - _E,), -1, jnp.int32)
    return (
        jnp.concatenate([e[0].astype(jnp.int32), pad]),
        jnp.concatenate([e[1].astype(jnp.int32), pad]),
    )


def kernel(x, users, bundles, edge_index_ub, edge_index_ui, edge_index_bi,
           emb, W1_ub, b1_ub, W1_ui, b1_ui, W1_bi, b1_bi,
           W2_ub, b2_ub, W2_ui, b2_ui, W2_bi, b2_bi,
           Wfc1, bfc1, Wfc2, bfc2, Wfc3, bfc3, Wout, bout):
    del x

    sf0, df0 = _pad_flat(edge_index_ub)
    sf1, df1 = _pad_flat(edge_index_ui)
    sf2, df2 = _pad_flat(edge_index_bi)

    dall = jnp.concatenate([
        jnp.where(df0 >= 0, df0 + _OFF[0], -1),
        jnp.where(df1 >= 0, df1 + _OFF[1], -1),
        jnp.where(df2 >= 0, df2 + _OFF[2], -1),
    ]).reshape((3 * _EPAD) // 128, 128)

    degs = _deg_call(dall)
    dinv_all = _dinv_call(degs.reshape(2, 784, 128)).reshape(-1)
    dv = (
        dinv_all[_OFF[0]:_OFF[0] + _N_UB].reshape(-1, 1),
        dinv_all[_OFF[1]:_OFF[1] + _N_UI].reshape(-1, 1),
        dinv_all[_OFF[2]:_OFF[2] + _N_BI].reshape(-1, 1),
    )

    x_ub = emb[:_N_UB]
    x_ui = jnp.concatenate([emb[:_NU], emb[_N_UB:]])
    x_bi = emb[_NU:]

    agg1 = _make_agg((_N_UB, _N_UI, _N_BI))
    agg2 = _make_agg((_N_UB, _NU, _NB))

    ws = ((W1_ub, W1_ui, W1_bi), (W2_ub, W2_ui, W2_bi))
    bs = (
        (b1_ub.reshape(1, _D), b1_ui.reshape(1, _D), b1_bi.reshape(1, _D)),
        (b2_ub.reshape(1, _D), b2_ui.reshape(1, _D), b2_bi.reshape(1, _D)),
    )

    h_u = h_b = None
    for layer in range(2):
        ht = tuple(
            _mm_scaled(xg, wg, dg)
            for xg, wg, dg in zip((x_ub, x_ui, x_bi), ws[layer], dv)
        )
        agg = agg1 if layer == 0 else agg2
        s = agg(ht[0], ht[1], ht[2], sf0, sf1, sf2, df0, df1, df2)
        b3_ = bs[layer]
        h_u = _combine(s[0], ht[0], dv[0], b3_[0], 0,
                       s[1], ht[1], dv[1], b3_[1], 0, _NU)
        h_b = _combine(s[0], ht[0], dv[0], b3_[0], _NU,
                       s[2], ht[2], dv[2], b3_[2], 0, _NB)
        if layer == 0:
            h_i = _combine(s[1], ht[1], dv[1], b3_[1], _NU,
                           s[2], ht[2], dv[2], b3_[2], _NB, _NI)
            x_ub = jnp.concatenate([h_u, h_b])
            x_ui = jnp.concatenate([h_u, h_i])
            x_bi = jnp.concatenate([h_b, h_i])

    iu2 = users.reshape(-1).astype(jnp.int32).reshape(128, 128)
    ib2 = (bundles.reshape(-1).astype(jnp.int32) - _NU).reshape(128, 128)
    zu, zb = _head_gather(h_u, h_b, iu2, ib2)

    return _mlp(
        zu, zb, Wfc1[:_D], Wfc1[_D:], bfc1.reshape(1, 64),
        Wfc2, bfc2.reshape(1, 32), Wfc3, bfc3.reshape(1, 16),
        Wout.reshape(1, 16), bout.reshape(1, 1),
    )
